# Initial kernel scaffold; baseline (speedup 1.0000x reference)
import jax
import jax.numpy as jnp
from jax.experimental import pallas as pl


def kernel(smpl_vertices, occ_volume, smpl_vertex_code, smpl_face_indices):
    raise NotImplementedError("write your pallas kernel here")


# speedup vs baseline: 97.7515x; 97.7515x over previous
import jax
import jax.numpy as jnp
from jax import lax
from jax.experimental import pallas as pl
from jax.experimental.pallas import tpu as pltpu

_VOL = 128
_VOX = 0.5 / 128.0
_INV2S2 = 1.0 / (2.0 * 0.005 * 0.005)
_R = 2.0
_IC = 8


def _dim_w(vt_ref, d, vpad):
    f32 = jnp.float32
    iw = lax.broadcasted_iota(jnp.int32, (_VOL, vpad), 0).astype(f32)
    c = vt_ref[0, d, :] * (1.0 / _VOX) + (_VOL / 2.0)
    c2 = jnp.broadcast_to(c[None, :], (_VOL, vpad))
    b = jnp.floor(c2)
    dist = (iw + 0.5 - c2) * _VOX
    w = jnp.exp(-(dist * dist) * _INV2S2)
    m = (iw >= b - _R) & (iw <= b + _R)
    return jnp.where(m, w, 0.0).astype(jnp.bfloat16)


def _splat_kernel(vt_ref, ct_ref, out_ref, ax_ref, ay_ref, q_ref):
    vpad = vt_ref.shape[2]
    ic = pl.program_id(1)

    @pl.when(ic == 0)
    def _prep():
        ax_ref[...] = _dim_w(vt_ref, 0, vpad)
        ay_ref[...] = _dim_w(vt_ref, 1, vpad)
        az = _dim_w(vt_ref, 2, vpad)
        for c in range(3):
            code_c = ct_ref[0, c, :].astype(jnp.bfloat16)
            q_ref[c * _VOL:(c + 1) * _VOL, :] = az * jnp.broadcast_to(
                code_c[None, :], (_VOL, vpad))
        q_ref[3 * _VOL:, :] = az

    axc = ax_ref[pl.ds(ic * _IC, _IC), :]
    ay = ay_ref[...]
    p = (axc[:, None, :] * ay[None, :, :]).reshape(_IC * _VOL, vpad)
    s = lax.dot_general(p, q_ref[...], (((1,), (1,)), ((), ())),
                        preferred_element_type=jnp.float32)
    s3 = s.reshape(_IC, _VOL, 4 * _VOL)
    wsum = s3[:, :, 3 * _VOL:] + 1e-3
    for c in range(3):
        out_ref[0, c, :, :, :] = s3[:, :, c * _VOL:(c + 1) * _VOL] / wsum


def kernel(smpl_vertices, occ_volume, smpl_vertex_code, smpl_face_indices):
    del occ_volume, smpl_face_indices
    bsz, nv, _ = smpl_vertices.shape
    vpad = ((nv + 127) // 128) * 128
    vt = jnp.transpose(smpl_vertices, (0, 2, 1))
    ct = jnp.transpose(smpl_vertex_code, (0, 2, 1))
    vt = jnp.pad(vt, ((0, 0), (0, 0), (0, vpad - nv)), constant_values=1e4)
    ct = jnp.pad(ct, ((0, 0), (0, 0), (0, vpad - nv)))
    out = pl.pallas_call(
        _splat_kernel,
        grid=(bsz, _VOL // _IC),
        in_specs=[
            pl.BlockSpec((1, 3, vpad), lambda b, i: (b, 0, 0)),
            pl.BlockSpec((1, 3, vpad), lambda b, i: (b, 0, 0)),
        ],
        out_specs=pl.BlockSpec((1, 3, _IC, _VOL, _VOL),
                               lambda b, i: (b, 0, i, 0, 0)),
        out_shape=jax.ShapeDtypeStruct((bsz, 3, _VOL, _VOL, _VOL), jnp.float32),
        scratch_shapes=[
            pltpu.VMEM((_VOL, vpad), jnp.bfloat16),
            pltpu.VMEM((_VOL, vpad), jnp.bfloat16),
            pltpu.VMEM((4 * _VOL, vpad), jnp.bfloat16),
        ],
        compiler_params=pltpu.CompilerParams(
            dimension_semantics=("arbitrary", "arbitrary"),
            vmem_limit_bytes=100 * 1024 * 1024,
        ),
    )(vt, ct)
    return out


# speedup vs baseline: 98.2918x; 1.0055x over previous
import jax
import jax.numpy as jnp
from jax import lax
from jax.experimental import pallas as pl
from jax.experimental.pallas import tpu as pltpu

_VOL = 128
_VOX = 0.5 / 128.0
_INV2S2 = 1.0 / (2.0 * 0.005 * 0.005)
_R = 2.0
_IC = 16


def _dim_w(vt_ref, d, vpad):
    f32 = jnp.float32
    iw = lax.broadcasted_iota(jnp.int32, (_VOL, vpad), 0).astype(f32)
    c = vt_ref[0, d, :] * (1.0 / _VOX) + (_VOL / 2.0)
    c2 = jnp.broadcast_to(c[None, :], (_VOL, vpad))
    b = jnp.floor(c2)
    dist = (iw + 0.5 - c2) * _VOX
    w = jnp.exp(-(dist * dist) * _INV2S2)
    m = (iw >= b - _R) & (iw <= b + _R)
    return jnp.where(m, w, 0.0).astype(jnp.bfloat16)


def _splat_kernel(vt_ref, ct_ref, out_ref, ax_ref, ay_ref, q_ref):
    vpad = vt_ref.shape[2]
    ic = pl.program_id(1)

    @pl.when(ic == 0)
    def _prep():
        ax_ref[...] = _dim_w(vt_ref, 0, vpad)
        ay_ref[...] = _dim_w(vt_ref, 1, vpad)
        az = _dim_w(vt_ref, 2, vpad)
        for c in range(3):
            code_c = ct_ref[0, c, :].astype(jnp.bfloat16)
            q_ref[c * _VOL:(c + 1) * _VOL, :] = az * jnp.broadcast_to(
                code_c[None, :], (_VOL, vpad))
        q_ref[3 * _VOL:, :] = az

    axc = ax_ref[pl.ds(ic * _IC, _IC), :]
    ay = ay_ref[...]
    p = (axc[:, None, :] * ay[None, :, :]).reshape(_IC * _VOL, vpad)
    s = lax.dot_general(p, q_ref[...], (((1,), (1,)), ((), ())),
                        preferred_element_type=jnp.float32)
    s3 = s.reshape(_IC, _VOL, 4 * _VOL)
    rw = 1.0 / (s3[:, :, 3 * _VOL:] + 1e-3)
    for c in range(3):
        out_ref[0, c, :, :, :] = s3[:, :, c * _VOL:(c + 1) * _VOL] * rw


def kernel(smpl_vertices, occ_volume, smpl_vertex_code, smpl_face_indices):
    del occ_volume, smpl_face_indices
    bsz, nv, _ = smpl_vertices.shape
    vpad = ((nv + 127) // 128) * 128
    vt = jnp.transpose(smpl_vertices, (0, 2, 1))
    ct = jnp.transpose(smpl_vertex_code, (0, 2, 1))
    vt = jnp.pad(vt, ((0, 0), (0, 0), (0, vpad - nv)), constant_values=1e4)
    ct = jnp.pad(ct, ((0, 0), (0, 0), (0, vpad - nv)))
    out = pl.pallas_call(
        _splat_kernel,
        grid=(bsz, _VOL // _IC),
        in_specs=[
            pl.BlockSpec((1, 3, vpad), lambda b, i: (b, 0, 0)),
            pl.BlockSpec((1, 3, vpad), lambda b, i: (b, 0, 0)),
        ],
        out_specs=pl.BlockSpec((1, 3, _IC, _VOL, _VOL),
                               lambda b, i: (b, 0, i, 0, 0)),
        out_shape=jax.ShapeDtypeStruct((bsz, 3, _VOL, _VOL, _VOL), jnp.float32),
        scratch_shapes=[
            pltpu.VMEM((_VOL, vpad), jnp.bfloat16),
            pltpu.VMEM((_VOL, vpad), jnp.bfloat16),
            pltpu.VMEM((4 * _VOL, vpad), jnp.bfloat16),
        ],
        compiler_params=pltpu.CompilerParams(
            dimension_semantics=("arbitrary", "arbitrary"),
            vmem_limit_bytes=100 * 1024 * 1024,
        ),
    )(vt, ct)
    return out
